# Initial kernel scaffold; baseline (speedup 1.0000x reference)
import jax
import jax.numpy as jnp
from jax.experimental import pallas as pl


def kernel(edge_scores, dst, gain, bias):
    raise NotImplementedError("write your pallas kernel here")


# speedup vs baseline: 6.2345x; 6.2345x over previous
import functools

import jax
import jax.numpy as jnp
from jax import lax
from jax.experimental import pallas as pl
from jax.experimental.pallas import tpu as pltpu
from jax.experimental.pallas import tpu_sc as plsc

N_SEG = 100000
NP = 100096
BE = 1024
NW = 32
NT = 16
RSTRIPE = NP // NT

_mesh = plsc.VectorSubcoreMesh(core_axis_name="c", subcore_axis_name="s")
_SC_PARAMS = pltpu.CompilerParams(
    use_tc_tiling_on_sc=False, needs_layout_passes=False)


def _worker_block_count(w, nblocks):
    full = nblocks // NW
    extra = (w < (nblocks - full * NW)).astype(jnp.int32)
    return full + extra


def _phase_a(scores3, dst2, zeros4, ones4):
    nblocks = scores3.shape[0]
    h = scores3.shape[2]

    def body(scores_hbm, dst_hbm, zeros_hbm, ones_hbm,
             osum, osq, ocnt,
             tsum, tsq, tcnt, sco_v, sq_v, dst_v, ones_v):
        cid = lax.axis_index("c")
        sid = lax.axis_index("s")
        w = sid * 2 + cid
        rows = pl.ds(sid * RSTRIPE, RSTRIPE)
        pltpu.sync_copy(zeros_hbm.at[rows, :], tsum.at[rows, :])
        pltpu.sync_copy(zeros_hbm.at[rows, :], tsq.at[rows, :])
        pltpu.sync_copy(zeros_hbm.at[rows, :], tcnt.at[rows, :])
        pltpu.sync_copy(ones_hbm, ones_v)
        plsc.subcore_barrier()

        riota = lax.iota(jnp.int32, 16)
        cvec = lax.rem(riota, 4)
        rv0 = lax.div(riota, 4)

        def block_body(i, _):
            b = w + NW * i
            pltpu.sync_copy(scores_hbm.at[b], sco_v)
            pltpu.sync_copy(dst_hbm.at[b], dst_v)

            def sq_body(g, rv):
                s = plsc.load_gather(sco_v, [rv, cvec])
                plsc.store_scatter(sq_v, [rv, cvec], s * s)
                return rv + 4

            lax.fori_loop(0, BE * h // 64, sq_body, rv0, unroll=4)
            pltpu.sync_copy(sco_v, tsum.at[dst_v], add=True)
            pltpu.sync_copy(sq_v, tsq.at[dst_v], add=True)
            pltpu.sync_copy(ones_v, tcnt.at[dst_v], add=True)
            return 0

        lax.fori_loop(0, _worker_block_count(w, nblocks), block_body, 0)
        plsc.subcore_barrier()
        pltpu.sync_copy(tsum.at[rows, :], osum.at[cid, rows, :])
        pltpu.sync_copy(tsq.at[rows, :], osq.at[cid, rows, :])
        pltpu.sync_copy(tcnt.at[rows, :], ocnt.at[cid, rows, :])

    f = pl.kernel(
        body,
        out_type=[jax.ShapeDtypeStruct((2, NP, 4), jnp.float32)] * 3,
        mesh=_mesh,
        compiler_params=_SC_PARAMS,
        scratch_types=[
            pltpu.VMEM_SHARED((NP, 4), jnp.float32),
            pltpu.VMEM_SHARED((NP, 4), jnp.float32),
            pltpu.VMEM_SHARED((NP, 4), jnp.float32),
            pltpu.VMEM((BE, 4), jnp.float32),
            pltpu.VMEM((BE, 4), jnp.float32),
            pltpu.VMEM((BE,), jnp.int32),
            pltpu.VMEM((BE, 4), jnp.float32),
        ],
    )
    return f(scores3, dst2, zeros4, ones4)


def _phase_b_body(ps, pq, pc, g, b, oa, ob):
    s = ps[0] + ps[1]
    q = pq[0] + pq[1]
    c = pc[0] + pc[1]
    mean = s / jnp.maximum(c, 1.0)
    var = jnp.maximum(q - c * mean * mean, 0.0)
    std = jnp.sqrt(var / jnp.maximum(c, 1.0))
    inv = 1.0 / jnp.maximum(std, 1e-5)
    a = g[0:1, :] * inv
    oa[...] = a
    ob[...] = b[0:1, :] - mean * a


def _phase_b(psum, psq, pcnt, gvec, bvec):
    rows = NP * 4 // 128
    f = pl.pallas_call(
        _phase_b_body,
        out_shape=[jax.ShapeDtypeStruct((rows, 128), jnp.float32)] * 2,
    )
    return f(psum.reshape(2, rows, 128), psq.reshape(2, rows, 128),
             pcnt.reshape(2, rows, 128), gvec, bvec)


def _phase_c(scores3, dst2, atab, btab):
    nblocks = scores3.shape[0]
    h = scores3.shape[2]

    def body(scores_hbm, dst_hbm, a_hbm, b_hbm, out_hbm,
             ta, tb, sco_v, ga_v, gb_v, out_v, dst_v):
        cid = lax.axis_index("c")
        sid = lax.axis_index("s")
        w = sid * 2 + cid
        rows = pl.ds(sid * RSTRIPE, RSTRIPE)
        pltpu.sync_copy(a_hbm.at[rows, :], ta.at[rows, :])
        pltpu.sync_copy(b_hbm.at[rows, :], tb.at[rows, :])
        plsc.subcore_barrier()

        riota = lax.iota(jnp.int32, 16)
        cvec = lax.rem(riota, 4)
        rv0 = lax.div(riota, 4)

        def block_body(i, _):
            b = w + NW * i
            pltpu.sync_copy(scores_hbm.at[b], sco_v)
            pltpu.sync_copy(dst_hbm.at[b], dst_v)
            pltpu.sync_copy(ta.at[dst_v], ga_v)
            pltpu.sync_copy(tb.at[dst_v], gb_v)

            def fma_body(g, rv):
                s = plsc.load_gather(sco_v, [rv, cvec])
                av = plsc.load_gather(ga_v, [rv, cvec])
                bv = plsc.load_gather(gb_v, [rv, cvec])
                plsc.store_scatter(out_v, [rv, cvec], av * s + bv)
                return rv + 4

            lax.fori_loop(0, BE * h // 64, fma_body, rv0, unroll=4)
            pltpu.sync_copy(out_v, out_hbm.at[b])
            return 0

        lax.fori_loop(0, _worker_block_count(w, nblocks), block_body, 0)

    f = pl.kernel(
        body,
        out_type=jax.ShapeDtypeStruct((nblocks, BE, h), jnp.float32),
        mesh=_mesh,
        compiler_params=_SC_PARAMS,
        scratch_types=[
            pltpu.VMEM_SHARED((NP, 4), jnp.float32),
            pltpu.VMEM_SHARED((NP, 4), jnp.float32),
            pltpu.VMEM((BE, 4), jnp.float32),
            pltpu.VMEM((BE, 4), jnp.float32),
            pltpu.VMEM((BE, 4), jnp.float32),
            pltpu.VMEM((BE, 4), jnp.float32),
            pltpu.VMEM((BE,), jnp.int32),
        ],
    )
    return f(scores3, dst2, atab, btab)


def kernel(edge_scores, dst, gain, bias):
    e, h, _ = edge_scores.shape
    nblocks = e // BE
    scores3 = edge_scores.reshape(nblocks, BE, h)
    dst2 = dst.reshape(nblocks, BE)
    zeros4 = jnp.zeros((NP, 4), jnp.float32)
    ones4 = jnp.ones((BE, 4), jnp.float32)
    psum, psq, pcnt = _phase_a(scores3, dst2, zeros4, ones4)
    gvec = jnp.tile(gain.reshape(1, h), (1, 128 // h))
    bvec = jnp.tile(bias.reshape(1, h), (1, 128 // h))
    a2, b2 = _phase_b(psum, psq, pcnt, gvec, bvec)
    out3 = _phase_c(scores3, dst2,
                    a2.reshape(NP, 4), b2.reshape(NP, 4))
    return out3.reshape(e, h, 1)
